# baseline (speedup 1.0000x reference)
import functools

import jax
import jax.numpy as jnp
from jax import lax
from jax.experimental import pallas as pl
from jax.experimental.pallas import tpu as pltpu
from jax.experimental.pallas import tpu_sc as plsc

N = 10000
NP = 10240
NACC = 10000
D = 256
HW = 128
E = 160000
CK = 128
NCHUNK = 1344
NREAL = E // CK
EP = NCHUNK * CK
NS = 16
CPT = NCHUNK // NS
RPT = 624
RB = 10240
GRID_R = NP // RB

_mesh = plsc.VectorSubcoreMesh(core_axis_name="c", subcore_axis_name="s")


@functools.partial(
    pl.kernel,
    mesh=_mesh,
    compiler_params=pltpu.CompilerParams(needs_layout_passes=False),
    out_type=jax.ShapeDtypeStruct((2 * NS, NP), jnp.float32),
    scratch_types=[
        pltpu.VMEM((6, 2, CK), jnp.int32),
        pltpu.VMEM((6, 2, CK), jnp.int32),
        pltpu.VMEM((NP,), jnp.float32),
        pltpu.SemaphoreType.DMA,
        pltpu.SemaphoreType.DMA,
    ],
)
def _hist_sc(pe_hbm, hists_hbm, idxa, idxb, hist, sa, sb):
    c = lax.axis_index("c")
    s = lax.axis_index("s")
    w = s * 2 + c
    zero16 = jnp.zeros((16,), jnp.float32)
    ones16 = jnp.ones((16,), jnp.float32)

    @pl.loop(0, NP, step=16)
    def _(i):
        hist[pl.ds(i, 16)] = zero16

    cpw = NCHUNK // (2 * NS)
    base = w * cpw
    bufs = (idxa, idxb)
    sems = (sa, sb)

    def issue(gi, bi):
        pltpu.async_copy(pe_hbm.at[pl.ds(base + 6 * gi, 6)], bufs[bi], sems[bi])

    def wait(bi):
        pltpu.make_async_copy(pe_hbm.at[pl.ds(0, 6)], bufs[bi], sems[bi]).wait()

    issue(0, 0)

    @pl.loop(0, 4)
    def _(q):
        for bi in (0, 1):
            gi = 2 * q + bi

            @pl.when(gi < 7)
            def _():
                @pl.when(gi + 1 < 7)
                def _():
                    issue(gi + 1, 1 - bi)

                wait(bi)

                for kk in range(6):
                    ch = base + 6 * gi + kk

                    @pl.when(ch < NREAL)
                    def _():
                        @pl.loop(0, CK, step=16)
                        def _(j):
                            dsts = bufs[bi][kk, 1, pl.ds(j, 16)]
                            plsc.addupdate_scatter(hist, [dsts], ones16)

    pltpu.sync_copy(hist, hists_hbm.at[w])


def _dinv_body(h_ref, o_ref):
    deg = jnp.sum(h_ref[...], axis=0, keepdims=True) + 1.0
    o_ref[...] = lax.rsqrt(deg)


_dinv_tc = pl.pallas_call(
    _dinv_body,
    out_shape=jax.ShapeDtypeStruct((1, NP), jnp.float32),
)


def _leaky(v):
    return jnp.where(v >= 0, v, 0.1 * v)


def _mm0_body(x_ref, dinv_ref, w_ref, o0_ref, o1_ref):
    i = pl.program_id(0)
    row = lax.broadcasted_iota(jnp.int32, (RB, 1), 0)
    xb = jnp.where(row < N - i * RB, x_ref[...], 0.0)
    hn = jnp.dot(xb, w_ref[...], preferred_element_type=jnp.float32)
    hs = dinv_ref[...] * hn
    o0_ref[...] = hs[:, :HW]
    o1_ref[...] = hs[:, HW:]


_mm0 = pl.pallas_call(
    _mm0_body,
    grid=(GRID_R,),
    in_specs=[
        pl.BlockSpec((RB, D), lambda i: (i, 0)),
        pl.BlockSpec((RB, 1), lambda i: (i, 0)),
        pl.BlockSpec((D, D), lambda i: (0, 0)),
    ],
    out_specs=[
        pl.BlockSpec((RB, HW), lambda i: (i, 0)),
        pl.BlockSpec((RB, HW), lambda i: (i, 0)),
    ],
    out_shape=[jax.ShapeDtypeStruct((NP, HW), jnp.float32)] * 2,
)


def _mid_body(s0_ref, s1_ref, dinv_ref, b_ref, w_ref, o0_ref, o1_ref):
    dv = dinv_ref[...]
    b = b_ref[...]
    xl = _leaky(dv * s0_ref[...] + b[:, :HW])
    xr = _leaky(dv * s1_ref[...] + b[:, HW:])
    xb = jnp.concatenate([xl, xr], axis=1)
    i = pl.program_id(0)
    row = lax.broadcasted_iota(jnp.int32, (RB, 1), 0)
    xb = jnp.where(row < N - i * RB, xb, 0.0)
    hn = jnp.dot(xb, w_ref[...], preferred_element_type=jnp.float32)
    hs = dv * hn
    o0_ref[...] = hs[:, :HW]
    o1_ref[...] = hs[:, HW:]


_mid = pl.pallas_call(
    _mid_body,
    grid=(GRID_R,),
    in_specs=[
        pl.BlockSpec((RB, HW), lambda i: (i, 0)),
        pl.BlockSpec((RB, HW), lambda i: (i, 0)),
        pl.BlockSpec((RB, 1), lambda i: (i, 0)),
        pl.BlockSpec((1, D), lambda i: (0, 0)),
        pl.BlockSpec((D, D), lambda i: (0, 0)),
    ],
    out_specs=[
        pl.BlockSpec((RB, HW), lambda i: (i, 0)),
        pl.BlockSpec((RB, HW), lambda i: (i, 0)),
    ],
    out_shape=[jax.ShapeDtypeStruct((NP, HW), jnp.float32)] * 2,
)

RBF = 1000


def _fin_body(s0_ref, s1_ref, dinv_ref, b_ref, o_ref):
    dv = dinv_ref[...]
    b = b_ref[...]
    o_ref[:, :HW] = _leaky(dv * s0_ref[...] + b[:, :HW])
    o_ref[:, HW:] = _leaky(dv * s1_ref[...] + b[:, HW:])


_fin = pl.pallas_call(
    _fin_body,
    grid=(N // RBF,),
    in_specs=[
        pl.BlockSpec((RBF, HW), lambda i: (i, 0)),
        pl.BlockSpec((RBF, HW), lambda i: (i, 0)),
        pl.BlockSpec((RBF, 1), lambda i: (i, 0)),
        pl.BlockSpec((1, D), lambda i: (0, 0)),
    ],
    out_specs=pl.BlockSpec((RBF, D), lambda i: (i, 0)),
    out_shape=jax.ShapeDtypeStruct((N, D), jnp.float32),
)


@functools.partial(
    pl.kernel,
    mesh=_mesh,
    out_type=[jax.ShapeDtypeStruct((NP, HW), jnp.float32)] * 2,
    scratch_types=[
        pltpu.VMEM_SHARED((NACC, HW), jnp.float32),
        pltpu.VMEM((2, CK), jnp.int32),
        pltpu.VMEM((2, CK), jnp.int32),
        pltpu.VMEM((2, CK), jnp.int32),
        pltpu.VMEM((2, CK), jnp.int32),
        pltpu.VMEM((CK, HW), jnp.float32),
        pltpu.VMEM((CK, HW), jnp.float32),
        pltpu.VMEM((CK, HW), jnp.float32),
        pltpu.SemaphoreType.DMA,
        pltpu.SemaphoreType.DMA,
        pltpu.SemaphoreType.DMA,
        pltpu.SemaphoreType.DMA,
        pltpu.SemaphoreType.DMA,
        pltpu.SemaphoreType.DMA,
        pltpu.SemaphoreType.DMA,
        pltpu.SemaphoreType.DMA,
        pltpu.SemaphoreType.DMA,
        pltpu.SemaphoreType.DMA,
    ],
)
def _scatter_sc(h0_hbm, h1_hbm, pe_hbm, s0_hbm, s1_hbm,
                acc, idx0, idx1, idx2, idx3, rows0, rows1, rows2,
                sp0, sp1, sp2, sp3, sg0, sg1, sg2, ss0, ss1, ss2):
    c = lax.axis_index("c")
    t = lax.axis_index("s")
    idxs = (idx0, idx1, idx2, idx3)
    rows = (rows0, rows1, rows2)
    sp = (sp0, sp1, sp2, sp3)
    sg = (sg0, sg1, sg2)
    ss = (ss0, ss1, ss2)

    cbase = t * CPT

    def issue_idx(ci, buf, sem):
        pltpu.async_copy(pe_hbm.at[cbase + ci], buf, sem)

    def wait_idx(buf, sem):
        pltpu.make_async_copy(pe_hbm.at[0], buf, sem).wait()

    def issue_gather(buf, rb, sem):
        @pl.when(c == 0)
        def _():
            pltpu.async_copy(h0_hbm.at[buf.at[0]], rb, sem)

        @pl.when(c == 1)
        def _():
            pltpu.async_copy(h1_hbm.at[buf.at[0]], rb, sem)

    def wait_rows_dma(rb, sem):
        pltpu.make_async_copy(h0_hbm.at[pl.ds(0, CK)], rb, sem).wait()

    def issue_scatter(buf, rb, sem):
        pltpu.async_copy(rb, acc.at[buf.at[1]], sem, add=True)

    issue_idx(0, idx0, sp0)
    issue_idx(1, idx1, sp1)
    issue_idx(2, idx2, sp2)
    wait_idx(idx0, sp0)
    issue_gather(idx0, rows0, sg0)
    wait_idx(idx1, sp1)
    issue_gather(idx1, rows1, sg1)

    zbase = t * RPT
    tail0 = NACC - NS * RPT

    @pl.when(c == 0)
    def _():
        pltpu.sync_copy(h0_hbm.at[pl.ds(zbase, RPT)], acc.at[pl.ds(zbase, RPT)])

        @pl.when(t == NS - 1)
        def _():
            pltpu.sync_copy(h0_hbm.at[pl.ds(NS * RPT, tail0)],
                            acc.at[pl.ds(NS * RPT, tail0)])

    @pl.when(c == 1)
    def _():
        pltpu.sync_copy(h1_hbm.at[pl.ds(zbase, RPT)], acc.at[pl.ds(zbase, RPT)])

        @pl.when(t == NS - 1)
        def _():
            pltpu.sync_copy(h1_hbm.at[pl.ds(NS * RPT, tail0)],
                            acc.at[pl.ds(NS * RPT, tail0)])

    plsc.subcore_barrier()

    @pl.loop(0, CPT // 12)
    def _(g):
        for j in range(12):
            b = j % 3
            b2 = (b + 2) % 3
            ib = j % 4
            ib2 = (j + 2) % 4
            ib3 = (j + 3) % 4
            ch = 12 * g + j

            @pl.when(ch >= 1)
            def _():
                wait_rows_dma(rows[b2], ss[b2])

            @pl.when(ch + 3 < CPT)
            def _():
                issue_idx(ch + 3, idxs[ib3], sp[ib3])

            @pl.when(ch + 2 < CPT)
            def _():
                wait_idx(idxs[ib2], sp[ib2])
                issue_gather(idxs[ib2], rows[b2], sg[b2])

            wait_rows_dma(rows[b], sg[b])
            issue_scatter(idxs[ib], rows[b], ss[b])

    wait_rows_dma(rows[2], ss[2])

    plsc.subcore_barrier()

    obase = t * RPT
    tail = NACC - NS * RPT

    @pl.when(c == 0)
    def _():
        pltpu.sync_copy(acc.at[pl.ds(obase, RPT)], s0_hbm.at[pl.ds(obase, RPT)])

        @pl.when(t == NS - 1)
        def _():
            pltpu.sync_copy(acc.at[pl.ds(NS * RPT, tail)],
                            s0_hbm.at[pl.ds(NS * RPT, tail)])

    @pl.when(c == 1)
    def _():
        pltpu.sync_copy(acc.at[pl.ds(obase, RPT)], s1_hbm.at[pl.ds(obase, RPT)])

        @pl.when(t == NS - 1)
        def _():
            pltpu.sync_copy(acc.at[pl.ds(NS * RPT, tail)],
                            s1_hbm.at[pl.ds(NS * RPT, tail)])


def kernel(x, edge_index, W0, b0, W1, b1, W2, b2, W3, b3, W4, b4):
    src = edge_index[0].astype(jnp.int32)
    dst = edge_index[1].astype(jnp.int32)
    pad = EP - E
    pad_src = N + jnp.mod(jnp.arange(pad, dtype=jnp.int32), NP - N)
    pad_dst = jnp.mod(jnp.arange(pad, dtype=jnp.int32) * 37, N).astype(jnp.int32)
    src_p = jnp.concatenate([src, pad_src])
    dst_p = jnp.concatenate([dst, pad_dst])
    pe = jnp.stack(
        [src_p.reshape(NCHUNK, CK), dst_p.reshape(NCHUNK, CK)], axis=1
    )

    hists = _hist_sc(pe)
    dinv = _dinv_tc(hists).reshape(NP, 1)

    weights = (W0, W1, W2, W3, W4)
    biases = tuple(b.reshape(1, D) for b in (b0, b1, b2, b3, b4))

    h0, h1 = _mm0(x, dinv, weights[0])
    for i in range(1, 5):
        s0, s1 = _scatter_sc(h0, h1, pe)
        h0, h1 = _mid(s0, s1, dinv, biases[i - 1], weights[i])
    s0, s1 = _scatter_sc(h0, h1, pe)
    return _fin(s0, s1, dinv, biases[4])


# speedup vs baseline: 1.0307x; 1.0307x over previous
import functools

import jax
import jax.numpy as jnp
from jax import lax
from jax.experimental import pallas as pl
from jax.experimental.pallas import tpu as pltpu
from jax.experimental.pallas import tpu_sc as plsc

N = 10000
NP = 10240
NACC = 10000
D = 256
HW = 128
E = 160000
CK = 128
NCHUNK = 1344
NREAL = E // CK
EP = NCHUNK * CK
NS = 16
CPT = NCHUNK // NS
RPT = 624
RB = 5120
GRID_R = NP // RB

_mesh = plsc.VectorSubcoreMesh(core_axis_name="c", subcore_axis_name="s")


@functools.partial(
    pl.kernel,
    mesh=_mesh,
    compiler_params=pltpu.CompilerParams(needs_layout_passes=False),
    out_type=jax.ShapeDtypeStruct((2 * NS, NP), jnp.float32),
    scratch_types=[
        pltpu.VMEM((6, 2, CK), jnp.int32),
        pltpu.VMEM((6, 2, CK), jnp.int32),
        pltpu.VMEM((NP,), jnp.float32),
        pltpu.SemaphoreType.DMA,
        pltpu.SemaphoreType.DMA,
    ],
)
def _hist_sc(pe_hbm, hists_hbm, idxa, idxb, hist, sa, sb):
    c = lax.axis_index("c")
    s = lax.axis_index("s")
    w = s * 2 + c
    zero16 = jnp.zeros((16,), jnp.float32)
    ones16 = jnp.ones((16,), jnp.float32)

    @pl.loop(0, NP, step=16)
    def _(i):
        hist[pl.ds(i, 16)] = zero16

    cpw = NCHUNK // (2 * NS)
    base = w * cpw
    bufs = (idxa, idxb)
    sems = (sa, sb)

    def issue(gi, bi):
        pltpu.async_copy(pe_hbm.at[pl.ds(base + 6 * gi, 6)], bufs[bi], sems[bi])

    def wait(bi):
        pltpu.make_async_copy(pe_hbm.at[pl.ds(0, 6)], bufs[bi], sems[bi]).wait()

    issue(0, 0)

    @pl.loop(0, 4)
    def _(q):
        for bi in (0, 1):
            gi = 2 * q + bi

            @pl.when(gi < 7)
            def _():
                @pl.when(gi + 1 < 7)
                def _():
                    issue(gi + 1, 1 - bi)

                wait(bi)

                for kk in range(6):
                    ch = base + 6 * gi + kk

                    @pl.when(ch < NREAL)
                    def _():
                        @pl.loop(0, CK, step=16)
                        def _(j):
                            dsts = bufs[bi][kk, 1, pl.ds(j, 16)]
                            plsc.addupdate_scatter(hist, [dsts], ones16)

    pltpu.sync_copy(hist, hists_hbm.at[w])


def _dinv_body(h_ref, o_ref):
    deg = jnp.sum(h_ref[...], axis=0, keepdims=True) + 1.0
    o_ref[...] = lax.rsqrt(deg)


_dinv_tc = pl.pallas_call(
    _dinv_body,
    out_shape=jax.ShapeDtypeStruct((1, NP), jnp.float32),
)


def _leaky(v):
    return jnp.where(v >= 0, v, 0.1 * v)


def _mm0_body(x_ref, dinv_ref, w_ref, o0_ref, o1_ref):
    i = pl.program_id(0)
    row = lax.broadcasted_iota(jnp.int32, (RB, 1), 0)
    xb = jnp.where(row < N - i * RB, x_ref[...], 0.0)
    hn = jnp.dot(xb, w_ref[...], preferred_element_type=jnp.float32)
    hs = dinv_ref[...] * hn
    o0_ref[...] = hs[:, :HW]
    o1_ref[...] = hs[:, HW:]


_mm0 = pl.pallas_call(
    _mm0_body,
    grid=(GRID_R,),
    in_specs=[
        pl.BlockSpec((RB, D), lambda i: (i, 0)),
        pl.BlockSpec((RB, 1), lambda i: (i, 0)),
        pl.BlockSpec((D, D), lambda i: (0, 0)),
    ],
    out_specs=[
        pl.BlockSpec((RB, HW), lambda i: (i, 0)),
        pl.BlockSpec((RB, HW), lambda i: (i, 0)),
    ],
    out_shape=[jax.ShapeDtypeStruct((NP, HW), jnp.float32)] * 2,
)


def _mid_body(s0_ref, s1_ref, dinv_ref, b_ref, w_ref, o0_ref, o1_ref):
    dv = dinv_ref[...]
    b = b_ref[...]
    xl = _leaky(dv * s0_ref[...] + b[:, :HW])
    xr = _leaky(dv * s1_ref[...] + b[:, HW:])
    xb = jnp.concatenate([xl, xr], axis=1)
    i = pl.program_id(0)
    row = lax.broadcasted_iota(jnp.int32, (RB, 1), 0)
    xb = jnp.where(row < N - i * RB, xb, 0.0)
    hn = jnp.dot(xb, w_ref[...], preferred_element_type=jnp.float32)
    hs = dv * hn
    o0_ref[...] = hs[:, :HW]
    o1_ref[...] = hs[:, HW:]


_mid = pl.pallas_call(
    _mid_body,
    grid=(GRID_R,),
    in_specs=[
        pl.BlockSpec((RB, HW), lambda i: (i, 0)),
        pl.BlockSpec((RB, HW), lambda i: (i, 0)),
        pl.BlockSpec((RB, 1), lambda i: (i, 0)),
        pl.BlockSpec((1, D), lambda i: (0, 0)),
        pl.BlockSpec((D, D), lambda i: (0, 0)),
    ],
    out_specs=[
        pl.BlockSpec((RB, HW), lambda i: (i, 0)),
        pl.BlockSpec((RB, HW), lambda i: (i, 0)),
    ],
    out_shape=[jax.ShapeDtypeStruct((NP, HW), jnp.float32)] * 2,
)

RBF = 5000


def _fin_body(s0_ref, s1_ref, dinv_ref, b_ref, o_ref):
    dv = dinv_ref[...]
    b = b_ref[...]
    o_ref[:, :HW] = _leaky(dv * s0_ref[...] + b[:, :HW])
    o_ref[:, HW:] = _leaky(dv * s1_ref[...] + b[:, HW:])


_fin = pl.pallas_call(
    _fin_body,
    grid=(N // RBF,),
    in_specs=[
        pl.BlockSpec((RBF, HW), lambda i: (i, 0)),
        pl.BlockSpec((RBF, HW), lambda i: (i, 0)),
        pl.BlockSpec((RBF, 1), lambda i: (i, 0)),
        pl.BlockSpec((1, D), lambda i: (0, 0)),
    ],
    out_specs=pl.BlockSpec((RBF, D), lambda i: (i, 0)),
    out_shape=jax.ShapeDtypeStruct((N, D), jnp.float32),
)


@functools.partial(
    pl.kernel,
    mesh=_mesh,
    out_type=[jax.ShapeDtypeStruct((NP, HW), jnp.float32)] * 2,
    scratch_types=[
        pltpu.VMEM_SHARED((NACC, HW), jnp.float32),
        pltpu.VMEM((2, CK), jnp.int32),
        pltpu.VMEM((2, CK), jnp.int32),
        pltpu.VMEM((2, CK), jnp.int32),
        pltpu.VMEM((2, CK), jnp.int32),
        pltpu.VMEM((CK, HW), jnp.float32),
        pltpu.VMEM((CK, HW), jnp.float32),
        pltpu.VMEM((CK, HW), jnp.float32),
        pltpu.SemaphoreType.DMA,
        pltpu.SemaphoreType.DMA,
        pltpu.SemaphoreType.DMA,
        pltpu.SemaphoreType.DMA,
        pltpu.SemaphoreType.DMA,
        pltpu.SemaphoreType.DMA,
        pltpu.SemaphoreType.DMA,
        pltpu.SemaphoreType.DMA,
        pltpu.SemaphoreType.DMA,
        pltpu.SemaphoreType.DMA,
    ],
)
def _scatter_sc(h0_hbm, h1_hbm, pe_hbm, s0_hbm, s1_hbm,
                acc, idx0, idx1, idx2, idx3, rows0, rows1, rows2,
                sp0, sp1, sp2, sp3, sg0, sg1, sg2, ss0, ss1, ss2):
    c = lax.axis_index("c")
    t = lax.axis_index("s")
    idxs = (idx0, idx1, idx2, idx3)
    rows = (rows0, rows1, rows2)
    sp = (sp0, sp1, sp2, sp3)
    sg = (sg0, sg1, sg2)
    ss = (ss0, ss1, ss2)

    cbase = t * CPT

    def issue_idx(ci, buf, sem):
        pltpu.async_copy(pe_hbm.at[cbase + ci], buf, sem)

    def wait_idx(buf, sem):
        pltpu.make_async_copy(pe_hbm.at[0], buf, sem).wait()

    def issue_gather(buf, rb, sem):
        @pl.when(c == 0)
        def _():
            pltpu.async_copy(h0_hbm.at[buf.at[0]], rb, sem)

        @pl.when(c == 1)
        def _():
            pltpu.async_copy(h1_hbm.at[buf.at[0]], rb, sem)

    def wait_rows_dma(rb, sem):
        pltpu.make_async_copy(h0_hbm.at[pl.ds(0, CK)], rb, sem).wait()

    def issue_scatter(buf, rb, sem):
        pltpu.async_copy(rb, acc.at[buf.at[1]], sem, add=True)

    issue_idx(0, idx0, sp0)
    issue_idx(1, idx1, sp1)
    issue_idx(2, idx2, sp2)
    wait_idx(idx0, sp0)
    issue_gather(idx0, rows0, sg0)
    wait_idx(idx1, sp1)
    issue_gather(idx1, rows1, sg1)

    zbase = t * RPT
    tail0 = NACC - NS * RPT

    @pl.when(c == 0)
    def _():
        pltpu.sync_copy(h0_hbm.at[pl.ds(zbase, RPT)], acc.at[pl.ds(zbase, RPT)])

        @pl.when(t == NS - 1)
        def _():
            pltpu.sync_copy(h0_hbm.at[pl.ds(NS * RPT, tail0)],
                            acc.at[pl.ds(NS * RPT, tail0)])

    @pl.when(c == 1)
    def _():
        pltpu.sync_copy(h1_hbm.at[pl.ds(zbase, RPT)], acc.at[pl.ds(zbase, RPT)])

        @pl.when(t == NS - 1)
        def _():
            pltpu.sync_copy(h1_hbm.at[pl.ds(NS * RPT, tail0)],
                            acc.at[pl.ds(NS * RPT, tail0)])

    plsc.subcore_barrier()

    @pl.loop(0, CPT // 12)
    def _(g):
        for j in range(12):
            b = j % 3
            b2 = (b + 2) % 3
            ib = j % 4
            ib2 = (j + 2) % 4
            ib3 = (j + 3) % 4
            ch = 12 * g + j

            @pl.when(ch >= 1)
            def _():
                wait_rows_dma(rows[b2], ss[b2])

            @pl.when(ch + 3 < CPT)
            def _():
                issue_idx(ch + 3, idxs[ib3], sp[ib3])

            @pl.when(ch + 2 < CPT)
            def _():
                wait_idx(idxs[ib2], sp[ib2])
                issue_gather(idxs[ib2], rows[b2], sg[b2])

            wait_rows_dma(rows[b], sg[b])
            issue_scatter(idxs[ib], rows[b], ss[b])

    wait_rows_dma(rows[2], ss[2])

    plsc.subcore_barrier()

    obase = t * RPT
    tail = NACC - NS * RPT

    @pl.when(c == 0)
    def _():
        pltpu.sync_copy(acc.at[pl.ds(obase, RPT)], s0_hbm.at[pl.ds(obase, RPT)])

        @pl.when(t == NS - 1)
        def _():
            pltpu.sync_copy(acc.at[pl.ds(NS * RPT, tail)],
                            s0_hbm.at[pl.ds(NS * RPT, tail)])

    @pl.when(c == 1)
    def _():
        pltpu.sync_copy(acc.at[pl.ds(obase, RPT)], s1_hbm.at[pl.ds(obase, RPT)])

        @pl.when(t == NS - 1)
        def _():
            pltpu.sync_copy(acc.at[pl.ds(NS * RPT, tail)],
                            s1_hbm.at[pl.ds(NS * RPT, tail)])


def kernel(x, edge_index, W0, b0, W1, b1, W2, b2, W3, b3, W4, b4):
    src = edge_index[0].astype(jnp.int32)
    dst = edge_index[1].astype(jnp.int32)
    pad = EP - E
    pad_src = N + jnp.mod(jnp.arange(pad, dtype=jnp.int32), NP - N)
    pad_dst = jnp.mod(jnp.arange(pad, dtype=jnp.int32) * 37, N).astype(jnp.int32)
    src_p = jnp.concatenate([src, pad_src])
    dst_p = jnp.concatenate([dst, pad_dst])
    pe = jnp.stack(
        [src_p.reshape(NCHUNK, CK), dst_p.reshape(NCHUNK, CK)], axis=1
    )

    hists = _hist_sc(pe)
    dinv = _dinv_tc(hists).reshape(NP, 1)

    weights = (W0, W1, W2, W3, W4)
    biases = tuple(b.reshape(1, D) for b in (b0, b1, b2, b3, b4))

    h0, h1 = _mm0(x, dinv, weights[0])
    for i in range(1, 5):
        s0, s1 = _scatter_sc(h0, h1, pe)
        h0, h1 = _mid(s0, s1, dinv, biases[i - 1], weights[i])
    s0, s1 = _scatter_sc(h0, h1, pe)
    return _fin(s0, s1, dinv, biases[4])
